# baseline (speedup 1.0000x reference)
import functools

import jax
import jax.numpy as jnp
from jax import lax
from jax.experimental import pallas as pl
from jax.experimental.pallas import tpu as pltpu
from jax.experimental.pallas import tpu_sc as plsc

VOCAB_SIZE = 1000000
EMBED_DIM = 64
BATCH = 16384

_INFO = plsc.get_sparse_core_info()
_NC, _NS, _L = _INFO.num_cores, _INFO.num_subcores, _INFO.num_lanes
_NW = _NC * _NS
_BPW = BATCH // _NW
_TS = 8
_CH = _L
_NCHUNK = _BPW // _CH
_KCH = EMBED_DIM // _L


def _sc_kernel(cidx_hbm, xidx_hbm, ctab_hbm, xtab_hbm, out_hbm,
               cidx_v, xidx_v, cbuf_e, cbuf_o, xbuf_e, xbuf_o, out_v,
               sem_i, sem_ce, sem_co, sem_xe, sem_xo):
    wid = lax.axis_index("s") * _NC + lax.axis_index("c")
    base = wid * _BPW

    cp_i = pltpu.async_copy(cidx_hbm.at[pl.ds(base, _BPW)], cidx_v, sem_i)
    cp_j = pltpu.async_copy(xidx_hbm.at[pl.ds(base, _BPW)], xidx_v, sem_i)
    cp_i.wait()
    cp_j.wait()

    lanes = lax.iota(jnp.int32, _L)

    def issue(c, cbuf, xbuf, sem_c, sem_x):
        rc_vec = cidx_v[pl.ds(c * _CH, _L)] & ~7
        rx_vec = xidx_v[pl.ds(c * _CH, _L)] & ~7
        for t in range(_CH):
            rc = pl.multiple_of(rc_vec[t], _TS)
            rx = pl.multiple_of(rx_vec[t], _TS)
            pltpu.async_copy(ctab_hbm.at[pl.ds(rc, _TS), :], cbuf.at[t],
                             sem_c)
            pltpu.async_copy(xtab_hbm.at[pl.ds(rx, _TS), :], xbuf.at[t],
                             sem_x)

    def wait(cbuf, xbuf, sem_c, sem_x):
        for t in range(_CH):
            pltpu.make_async_copy(ctab_hbm.at[pl.ds(0, _TS), :],
                                  cbuf.at[t], sem_c).wait()
            pltpu.make_async_copy(xtab_hbm.at[pl.ds(0, _TS), :],
                                  xbuf.at[t], sem_x).wait()

    def lane_sum(v):
        for sh in (8, 4, 2, 1):
            v = v + jnp.take_along_axis(v, lanes ^ sh, axis=0,
                                        mode="promise_in_bounds")
        return v

    def compute(c, cbuf, xbuf):
        r0 = c * _CH
        csub = cidx_v[pl.ds(r0, _L)] & 7
        xsub = xidx_v[pl.ds(r0, _L)] & 7
        tot = jnp.zeros((_L,), jnp.float32)
        for t in range(_CH):
            sc = csub[t]
            sx = xsub[t]
            acc = cbuf[t, sc, pl.ds(0, _L)] * xbuf[t, sx, pl.ds(0, _L)]
            for k in range(1, _KCH):
                acc = acc + (cbuf[t, sc, pl.ds(k * _L, _L)]
                             * xbuf[t, sx, pl.ds(k * _L, _L)])
            tot = jnp.where(lanes == t, lane_sum(acc), tot)
        out_v[pl.ds(r0, _L)] = tot

    issue(0, cbuf_e, xbuf_e, sem_ce, sem_xe)
    issue(1, cbuf_o, xbuf_o, sem_co, sem_xo)

    def pair(j, carry):
        c_even = j * 2

        wait(cbuf_e, xbuf_e, sem_ce, sem_xe)
        compute(c_even, cbuf_e, xbuf_e)

        @pl.when(c_even + 2 < _NCHUNK)
        def _prefetch_even():
            issue(c_even + 2, cbuf_e, xbuf_e, sem_ce, sem_xe)

        wait(cbuf_o, xbuf_o, sem_co, sem_xo)
        compute(c_even + 1, cbuf_o, xbuf_o)

        @pl.when(c_even + 3 < _NCHUNK)
        def _prefetch_odd():
            issue(c_even + 3, cbuf_o, xbuf_o, sem_co, sem_xo)

        return carry

    lax.fori_loop(0, _NCHUNK // 2, pair, 0)

    pltpu.sync_copy(out_v, out_hbm.at[pl.ds(base, _BPW)])


def kernel(center_word_idx, context_word_idx, center_embeddings,
           context_embeddings):
    mesh = plsc.VectorSubcoreMesh(core_axis_name="c", subcore_axis_name="s")
    k = functools.partial(
        pl.kernel,
        mesh=mesh,
        out_type=jax.ShapeDtypeStruct((BATCH,), jnp.float32),
        scratch_types=[
            pltpu.VMEM((_BPW,), jnp.int32),
            pltpu.VMEM((_BPW,), jnp.int32),
            pltpu.VMEM((_CH, _TS, EMBED_DIM), jnp.float32),
            pltpu.VMEM((_CH, _TS, EMBED_DIM), jnp.float32),
            pltpu.VMEM((_CH, _TS, EMBED_DIM), jnp.float32),
            pltpu.VMEM((_CH, _TS, EMBED_DIM), jnp.float32),
            pltpu.VMEM((_BPW,), jnp.float32),
            pltpu.SemaphoreType.DMA,
            pltpu.SemaphoreType.DMA,
            pltpu.SemaphoreType.DMA,
            pltpu.SemaphoreType.DMA,
            pltpu.SemaphoreType.DMA,
        ],
    )(_sc_kernel)
    return k(center_word_idx.astype(jnp.int32),
             context_word_idx.astype(jnp.int32),
             center_embeddings, context_embeddings)


# speedup vs baseline: 1.4817x; 1.4817x over previous
import functools

import jax
import jax.numpy as jnp
from jax import lax
from jax.experimental import pallas as pl
from jax.experimental.pallas import tpu as pltpu
from jax.experimental.pallas import tpu_sc as plsc

VOCAB_SIZE = 1000000
EMBED_DIM = 64
BATCH = 16384

_INFO = plsc.get_sparse_core_info()
_NC, _NS, _L = _INFO.num_cores, _INFO.num_subcores, _INFO.num_lanes
_NW = _NC * _NS
_BPW = BATCH // _NW
_TS = 8
_CH = _L
_NCHUNK = _BPW // _CH
_KCH = EMBED_DIM // _L


def _sc_kernel(cidx_hbm, xidx_hbm, ctab_hbm, xtab_hbm, out_hbm,
               cidx_v, xidx_v, cbuf_e, cbuf_o, xbuf_e, xbuf_o, out_v,
               sem_i, sem_ce, sem_co, sem_xe, sem_xo):
    wid = lax.axis_index("s") * _NC + lax.axis_index("c")
    base = wid * _BPW

    cp_i = pltpu.async_copy(cidx_hbm.at[pl.ds(base, _BPW)], cidx_v, sem_i)
    cp_j = pltpu.async_copy(xidx_hbm.at[pl.ds(base, _BPW)], xidx_v, sem_i)
    cp_i.wait()
    cp_j.wait()

    lanes = lax.iota(jnp.int32, _L)

    def issue(c, cbuf, xbuf, sem_c, sem_x):
        rc_vec = jnp.right_shift(cidx_v[pl.ds(c * _CH, _L)], 3)
        rx_vec = jnp.right_shift(xidx_v[pl.ds(c * _CH, _L)], 3)
        for t in range(_CH):
            pltpu.async_copy(ctab_hbm.at[rc_vec[t]], cbuf.at[t], sem_c)
            pltpu.async_copy(xtab_hbm.at[rx_vec[t]], xbuf.at[t], sem_x)

    def wait(cbuf, xbuf, sem_c, sem_x):
        pltpu.make_async_copy(ctab_hbm.at[pl.ds(0, _CH)], cbuf, sem_c).wait()
        pltpu.make_async_copy(xtab_hbm.at[pl.ds(0, _CH)], xbuf, sem_x).wait()

    def lane_sum(v):
        for sh in (8, 4, 2, 1):
            v = v + jnp.take_along_axis(v, lanes ^ sh, axis=0,
                                        mode="promise_in_bounds")
        return v

    def compute(c, cbuf, xbuf):
        r0 = c * _CH
        csub = cidx_v[pl.ds(r0, _L)] & 7
        xsub = xidx_v[pl.ds(r0, _L)] & 7
        tot = jnp.zeros((_L,), jnp.float32)
        for t in range(_CH):
            sc = csub[t]
            sx = xsub[t]
            acc = cbuf[t, sc, pl.ds(0, _L)] * xbuf[t, sx, pl.ds(0, _L)]
            for k in range(1, _KCH):
                acc = acc + (cbuf[t, sc, pl.ds(k * _L, _L)]
                             * xbuf[t, sx, pl.ds(k * _L, _L)])
            tot = jnp.where(lanes == t, lane_sum(acc), tot)
        out_v[pl.ds(r0, _L)] = tot

    issue(0, cbuf_e, xbuf_e, sem_ce, sem_xe)
    issue(1, cbuf_o, xbuf_o, sem_co, sem_xo)

    def pair(j, carry):
        c_even = j * 2

        wait(cbuf_e, xbuf_e, sem_ce, sem_xe)
        compute(c_even, cbuf_e, xbuf_e)

        @pl.when(c_even + 2 < _NCHUNK)
        def _prefetch_even():
            issue(c_even + 2, cbuf_e, xbuf_e, sem_ce, sem_xe)

        wait(cbuf_o, xbuf_o, sem_co, sem_xo)
        compute(c_even + 1, cbuf_o, xbuf_o)

        @pl.when(c_even + 3 < _NCHUNK)
        def _prefetch_odd():
            issue(c_even + 3, cbuf_o, xbuf_o, sem_co, sem_xo)

        return carry

    lax.fori_loop(0, _NCHUNK // 2, pair, 0)

    pltpu.sync_copy(out_v, out_hbm.at[pl.ds(base, _BPW)])


def kernel(center_word_idx, context_word_idx, center_embeddings,
           context_embeddings):
    ctab3 = center_embeddings.reshape(VOCAB_SIZE // _TS, _TS, EMBED_DIM)
    xtab3 = context_embeddings.reshape(VOCAB_SIZE // _TS, _TS, EMBED_DIM)
    mesh = plsc.VectorSubcoreMesh(core_axis_name="c", subcore_axis_name="s")
    k = functools.partial(
        pl.kernel,
        mesh=mesh,
        out_type=jax.ShapeDtypeStruct((BATCH,), jnp.float32),
        scratch_types=[
            pltpu.VMEM((_BPW,), jnp.int32),
            pltpu.VMEM((_BPW,), jnp.int32),
            pltpu.VMEM((_CH, _TS, EMBED_DIM), jnp.float32),
            pltpu.VMEM((_CH, _TS, EMBED_DIM), jnp.float32),
            pltpu.VMEM((_CH, _TS, EMBED_DIM), jnp.float32),
            pltpu.VMEM((_CH, _TS, EMBED_DIM), jnp.float32),
            pltpu.VMEM((_BPW,), jnp.float32),
            pltpu.SemaphoreType.DMA,
            pltpu.SemaphoreType.DMA,
            pltpu.SemaphoreType.DMA,
            pltpu.SemaphoreType.DMA,
            pltpu.SemaphoreType.DMA,
        ],
    )(_sc_kernel)
    return k(center_word_idx.astype(jnp.int32),
             context_word_idx.astype(jnp.int32),
             ctab3, xtab3)
